# baseline (speedup 1.0000x reference)
import functools

import jax
import jax.numpy as jnp
from jax import lax
from jax.experimental import pallas as pl
from jax.experimental.pallas import tpu as pltpu
from jax.experimental.pallas import tpu_sc as plsc

VOCAB = 30522
D = 768
B = 16
L = 512

NC = 2
NS = 16
NW = NC * NS
TOK = B * L
CH = TOK // NW
PW = L // CH
P = 32
NP = CH // P
NBUF = 4
DV = D // 16


def _body(ids_hbm, len_hbm, table_hbm, out_hbm,
          idx_refs, slv, buf_refs, isem, gsems, ssems):
    wid = lax.axis_index("s") * NC + lax.axis_index("c")
    b = wid // PW
    l_start = (wid % PW) * CH

    def stage_idx(i):
        return pltpu.make_async_copy(
            ids_hbm.at[b, pl.ds(l_start + i * P, P)], idx_refs[i], isem)

    def gather(i, s):
        pltpu.make_async_copy(
            table_hbm.at[idx_refs[i]], buf_refs[s], gsems[s]).start()

    def scat(i, s):
        return pltpu.make_async_copy(
            buf_refs[s], out_hbm.at[b, pl.ds(l_start + i * P, P)], ssems[s])

    stage_idx(0).start()
    stage_idx(1).start()
    stage_idx(0).wait()
    gather(0, 0)
    stage_idx(1).wait()
    gather(1, 1)
    for i in range(2, NP):
        stage_idx(i).start()

    pltpu.sync_copy(len_hbm, slv)
    for i in range(2, NP):
        stage_idx(i).wait()

    lane = lax.broadcasted_iota(jnp.int32, (16,), 0)
    sl = jnp.max(jnp.where(lane == b, slv[...], 0))
    nv = lax.max(lax.min(sl - l_start, CH), 0)

    zeros16 = jnp.zeros((16,), jnp.float32)

    for j in range(NP):
        s = j % NBUF
        buf = buf_refs[s]
        lo = lax.max(lax.min(nv - j * P, P), 0)

        pltpu.make_async_copy(
            table_hbm.at[idx_refs[j]], buf, gsems[s]).wait()

        def zo(r, _):
            for c in range(DV):
                buf[r, pl.ds(c * 16, 16)] = zeros16
            return 0

        lax.fori_loop(lo, P, zo, 0)

        scat(j, s).start()

        if j + 2 < NP:
            s2 = (j + 2) % NBUF
            if j - 2 >= 0:
                scat(j - 2, s2).wait()
            gather(j + 2, s2)

    for j in range(max(NP - NBUF, 0), NP):
        scat(j, j % NBUF).wait()


@functools.partial(jax.jit, static_argnames=())
def kernel(seq_ids, seq_len, table):
    def body(ids_hbm, len_hbm, table_hbm, out_hbm, *rest):
        idx_refs = rest[:NP]
        slv = rest[NP]
        buf_refs = rest[NP + 1:NP + 1 + NBUF]
        isem = rest[NP + 1 + NBUF]
        gsems = rest[NP + 2 + NBUF:NP + 2 + 2 * NBUF]
        ssems = rest[NP + 2 + 2 * NBUF:]
        _body(ids_hbm, len_hbm, table_hbm, out_hbm,
              idx_refs, slv, buf_refs, isem, gsems, ssems)

    run = pl.kernel(
        body,
        out_type=jax.ShapeDtypeStruct((B, L, D), jnp.float32),
        mesh=plsc.VectorSubcoreMesh(core_axis_name="c", subcore_axis_name="s"),
        compiler_params=pltpu.CompilerParams(needs_layout_passes=False),
        scratch_types=(
            [pltpu.VMEM((P,), jnp.int32) for _ in range(NP)]
            + [pltpu.VMEM((16,), jnp.int32)]
            + [pltpu.VMEM((P, D), jnp.float32) for _ in range(NBUF)]
            + [pltpu.SemaphoreType.DMA]
            + [pltpu.SemaphoreType.DMA for _ in range(2 * NBUF)]
        ),
    )
    return run(seq_ids, seq_len, table)


# speedup vs baseline: 1.0553x; 1.0553x over previous
import functools

import jax
import jax.numpy as jnp
from jax import lax
from jax.experimental import pallas as pl
from jax.experimental.pallas import tpu as pltpu
from jax.experimental.pallas import tpu_sc as plsc

VOCAB = 30522
D = 768
B = 16
L = 512

NC = 2
NS = 16
NW = NC * NS
TOK = B * L
CH = TOK // NW
PW = L // CH
P = 64
NP = CH // P
G = 16
DV = D // 16


def _body(ids_hbm, len_hbm, table_hbm, out_hbm,
          idx_refs, slv, bufA, bufB, zbuf, isem, zsem, gsems, ssems):
    wid = lax.axis_index("s") * NC + lax.axis_index("c")
    b = wid // PW
    l_start = (wid % PW) * CH

    bufs = (bufA, bufB)

    def stage_idx(i):
        return pltpu.make_async_copy(
            ids_hbm.at[b, pl.ds(l_start + i * P, P)], idx_refs[i], isem)

    def gather(i, s):
        return pltpu.make_async_copy(
            table_hbm.at[idx_refs[i]], bufs[s], gsems[s])

    stage_idx(0).start()
    stage_idx(1).start()
    stage_idx(0).wait()
    gather(0, 0).start()
    stage_idx(1).wait()
    gather(1, 1).start()
    for i in range(2, NP):
        stage_idx(i).start()

    pltpu.sync_copy(len_hbm, slv)
    for i in range(2, NP):
        stage_idx(i).wait()

    lane = lax.broadcasted_iota(jnp.int32, (16,), 0)
    sl = jnp.max(jnp.where(lane == b, slv[...], 0))
    nv = lax.max(lax.min(sl - l_start, CH), 0)

    zeros16 = jnp.zeros((16,), jnp.float32)

    def zrow(r, _):
        for c in range(DV):
            zbuf[r, pl.ds(c * 16, 16)] = zeros16
        return 0

    lax.fori_loop(0, G, zrow, 0)

    ztot = jnp.int32(0)

    for j in range(NP):
        s = j & 1
        buf = bufs[s]
        row0 = l_start + j * P
        lo = lax.max(lax.min(nv - j * P, P), 0)
        a16 = (lo + (G - 1)) & ~(G - 1)
        nu = a16 // G
        nz = (P - a16) // G

        def zfill(i, t):
            pltpu.make_async_copy(
                zbuf,
                out_hbm.at[b, pl.ds(pl.multiple_of(row0 + a16 + i * G, G), G)],
                zsem
            ).start()
            return t + 1

        ztot = lax.fori_loop(0, nz, zfill, ztot)

        if j < 2:
            gather(j, s).wait()
        else:
            def gwait(i, _):
                gather(j, s).wait()
                return 0

            lax.fori_loop(0, lax.min(nu, 1), gwait, 0)

        def zo(r, _):
            for c in range(DV):
                buf[r, pl.ds(c * 16, 16)] = zeros16
            return 0

        lax.fori_loop(lo, a16, zo, 0)

        def wb(i, _):
            pltpu.make_async_copy(
                buf.at[pl.ds(pl.multiple_of(i * G, G), G)],
                out_hbm.at[b, pl.ds(pl.multiple_of(row0 + i * G, G), G)], ssems[s]
            ).start()
            return 0

        lax.fori_loop(0, nu, wb, 0)

        if j + 2 < NP:
            def swait(i, _):
                pltpu.make_async_copy(
                    buf.at[pl.ds(pl.multiple_of(i * G, G), G)],
                    out_hbm.at[b, pl.ds(pl.multiple_of(row0 + i * G, G), G)], ssems[s]
                ).wait()
                return 0

            lax.fori_loop(0, nu, swait, 0)

            lo2 = lax.max(lax.min(nv - (j + 2) * P, P), 0)
            nu2 = lax.min((lo2 + (G - 1)) // G, 1)

            def gfire(i, _):
                gather(j + 2, s).start()
                return 0

            lax.fori_loop(0, nu2, gfire, 0)

    for j in (NP - 2, NP - 1):
        s = j & 1
        row0 = l_start + j * P
        lo = lax.max(lax.min(nv - j * P, P), 0)
        nu = ((lo + (G - 1)) & ~(G - 1)) // G

        def swait2(i, _):
            pltpu.make_async_copy(
                bufs[s].at[pl.ds(pl.multiple_of(i * G, G), G)],
                out_hbm.at[b, pl.ds(pl.multiple_of(row0 + i * G, G), G)], ssems[s]
            ).wait()
            return 0

        lax.fori_loop(0, nu, swait2, 0)

    def zdrain(i, _):
        pltpu.make_async_copy(
            zbuf, out_hbm.at[b, pl.ds(l_start, G)], zsem).wait()
        return 0

    lax.fori_loop(0, ztot, zdrain, 0)


@functools.partial(jax.jit, static_argnames=())
def kernel(seq_ids, seq_len, table):
    def body(ids_hbm, len_hbm, table_hbm, out_hbm, *rest):
        idx_refs = rest[:NP]
        slv = rest[NP]
        bufA, bufB, zbuf = rest[NP + 1:NP + 4]
        isem, zsem = rest[NP + 4:NP + 6]
        gsems = rest[NP + 6:NP + 8]
        ssems = rest[NP + 8:NP + 10]
        _body(ids_hbm, len_hbm, table_hbm, out_hbm,
              idx_refs, slv, bufA, bufB, zbuf, isem, zsem, gsems, ssems)

    run = pl.kernel(
        body,
        out_type=jax.ShapeDtypeStruct((B, L, D), jnp.float32),
        mesh=plsc.VectorSubcoreMesh(core_axis_name="c", subcore_axis_name="s"),
        compiler_params=pltpu.CompilerParams(needs_layout_passes=False),
        scratch_types=(
            [pltpu.VMEM((P,), jnp.int32) for _ in range(NP)]
            + [pltpu.VMEM((16,), jnp.int32)]
            + [pltpu.VMEM((P, D), jnp.float32) for _ in range(2)]
            + [pltpu.VMEM((G, D), jnp.float32)]
            + [pltpu.SemaphoreType.DMA for _ in range(6)]
        ),
    )
    return run(seq_ids, seq_len, table)


# speedup vs baseline: 1.0926x; 1.0353x over previous
import functools

import jax
import jax.numpy as jnp
from jax import lax
from jax.experimental import pallas as pl
from jax.experimental.pallas import tpu as pltpu
from jax.experimental.pallas import tpu_sc as plsc

VOCAB = 30522
D = 768
B = 16
L = 512

NC = 2
NS = 16
NW = NC * NS
TOK = B * L
CH = TOK // NW
PW = L // CH
P = 64
NP = CH // P
G = 16
DV = D // 16


def _body(ids_hbm, len_hbm, table_hbm, out_hbm,
          idx_refs, slv, bufA, bufB, zbuf, isem, zsem, gsems, ssems):
    wid = lax.axis_index("s") * NC + lax.axis_index("c")
    b = wid // PW
    l_start = (wid % PW) * CH

    bufs = (bufA, bufB)

    def stage_idx(i):
        return pltpu.make_async_copy(
            ids_hbm.at[b, pl.ds(l_start + i * P, P)], idx_refs[i], isem)

    def gather(i, s):
        return pltpu.make_async_copy(
            table_hbm.at[idx_refs[i]], bufs[s], gsems[s])

    stage_idx(0).start()
    stage_idx(1).start()
    stage_idx(0).wait()
    gather(0, 0).start()
    stage_idx(1).wait()
    gather(1, 1).start()
    for i in range(2, NP):
        stage_idx(i).start()

    pltpu.sync_copy(len_hbm, slv)
    for i in range(2, NP):
        stage_idx(i).wait()

    lane = lax.broadcasted_iota(jnp.int32, (16,), 0)
    sl = jnp.max(jnp.where(lane == b, slv[...], 0))
    nv = lax.max(lax.min(sl - l_start, CH), 0)

    zeros16 = jnp.zeros((16,), jnp.float32)

    def zrow(r, _):
        for c in range(DV):
            zbuf[r, pl.ds(c * 16, 16)] = zeros16
        return 0

    lax.fori_loop(0, G, zrow, 0)

    ztot = jnp.int32(0)

    for j in range(NP):
        s = j & 1
        buf = bufs[s]
        row0 = l_start + j * P
        lo = lax.max(lax.min(nv - j * P, P), 0)
        a16 = (lo + (G - 1)) & ~(G - 1)
        nu = a16 // G
        nz = (P - a16) // G

        def zfill(i, t):
            pltpu.make_async_copy(
                zbuf,
                out_hbm.at[b, pl.ds(pl.multiple_of(row0 + a16 + i * G, G), G)],
                zsem
            ).start()
            return t + 1

        ztot = lax.fori_loop(0, nz, zfill, ztot)

        if j < 2:
            gather(j, s).wait()
        else:
            def gwait(i, _):
                gather(j, s).wait()
                return 0

            lax.fori_loop(0, lax.min(nu, 1), gwait, 0)

        def zo(k, _):
            buf[lo + k // DV, pl.ds((k % DV) * 16, 16)] = zeros16
            return 0

        lax.fori_loop(0, (a16 - lo) * DV, zo, 0)

        def wb(i, _):
            pltpu.make_async_copy(
                buf.at[pl.ds(pl.multiple_of(i * G, G), G)],
                out_hbm.at[b, pl.ds(pl.multiple_of(row0 + i * G, G), G)], ssems[s]
            ).start()
            return 0

        lax.fori_loop(0, nu, wb, 0)

        if j + 2 < NP:
            def swait(i, _):
                pltpu.make_async_copy(
                    buf.at[pl.ds(pl.multiple_of(i * G, G), G)],
                    out_hbm.at[b, pl.ds(pl.multiple_of(row0 + i * G, G), G)], ssems[s]
                ).wait()
                return 0

            lax.fori_loop(0, nu, swait, 0)

            lo2 = lax.max(lax.min(nv - (j + 2) * P, P), 0)
            nu2 = lax.min((lo2 + (G - 1)) // G, 1)

            def gfire(i, _):
                gather(j + 2, s).start()
                return 0

            lax.fori_loop(0, nu2, gfire, 0)

    for j in (NP - 2, NP - 1):
        s = j & 1
        row0 = l_start + j * P
        lo = lax.max(lax.min(nv - j * P, P), 0)
        nu = ((lo + (G - 1)) & ~(G - 1)) // G

        def swait2(i, _):
            pltpu.make_async_copy(
                bufs[s].at[pl.ds(pl.multiple_of(i * G, G), G)],
                out_hbm.at[b, pl.ds(pl.multiple_of(row0 + i * G, G), G)], ssems[s]
            ).wait()
            return 0

        lax.fori_loop(0, nu, swait2, 0)

    def zdrain(i, _):
        pltpu.make_async_copy(
            zbuf, out_hbm.at[b, pl.ds(l_start, G)], zsem).wait()
        return 0

    lax.fori_loop(0, ztot, zdrain, 0)


@functools.partial(jax.jit, static_argnames=())
def kernel(seq_ids, seq_len, table):
    def body(ids_hbm, len_hbm, table_hbm, out_hbm, *rest):
        idx_refs = rest[:NP]
        slv = rest[NP]
        bufA, bufB, zbuf = rest[NP + 1:NP + 4]
        isem, zsem = rest[NP + 4:NP + 6]
        gsems = rest[NP + 6:NP + 8]
        ssems = rest[NP + 8:NP + 10]
        _body(ids_hbm, len_hbm, table_hbm, out_hbm,
              idx_refs, slv, bufA, bufB, zbuf, isem, zsem, gsems, ssems)

    run = pl.kernel(
        body,
        out_type=jax.ShapeDtypeStruct((B, L, D), jnp.float32),
        mesh=plsc.VectorSubcoreMesh(core_axis_name="c", subcore_axis_name="s"),
        compiler_params=pltpu.CompilerParams(needs_layout_passes=False),
        scratch_types=(
            [pltpu.VMEM((P,), jnp.int32) for _ in range(NP)]
            + [pltpu.VMEM((16,), jnp.int32)]
            + [pltpu.VMEM((P, D), jnp.float32) for _ in range(2)]
            + [pltpu.VMEM((G, D), jnp.float32)]
            + [pltpu.SemaphoreType.DMA for _ in range(6)]
        ),
    )
    return run(seq_ids, seq_len, table)
